# baseline (speedup 1.0000x reference)
import functools

import jax
import jax.numpy as jnp
from jax import lax
from jax.experimental import pallas as pl
from jax.experimental.pallas import tpu as pltpu
from jax.experimental.pallas import tpu_sc as plsc

NNODES = 10000
NEDGES = 320000
DIN = 128
DHID = 16
NCLS = 64

NC, NS = 2, 16
NW = NC * NS
RPAD = 10240
EROWS = 2560
NREAL = NEDGES // 128
WROWS = EROWS // NW
WIN = 4
NWIN = WROWS // WIN
NBUF = 4
SLC = RPAD // NS

import numpy as _np
_pi = _np.arange((EROWS - NREAL) * 128, dtype=_np.int32)
_EPAD_CONST = _np.stack([
    (_pi * 37) % NNODES,
    NNODES + _pi % (RPAD - NNODES),
]).reshape(2, EROWS - NREAL, 128)

_BLK = 512
_GRID = RPAD // _BLK


def _hist_body(edges, deg_out, idx_v, ones_v, zer_v, acc_sh, sem):
    cid = lax.axis_index("c")
    sid = lax.axis_index("s")
    wid = sid * NC + cid

    @pl.loop(0, 8)
    def _(i):
        ones_v[pl.ds(i * 16, 16)] = jnp.ones((16,), jnp.float32)

    @pl.loop(0, SLC // 16)
    def _(i):
        zer_v[pl.ds(i * 16, 16)] = jnp.zeros((16,), jnp.float32)

    pltpu.sync_copy(edges.at[1, pl.ds(wid * WROWS, WROWS)], idx_v)
    pltpu.sync_copy(zer_v, acc_sh.at[pl.ds(sid * SLC, SLC)])
    plsc.subcore_barrier()

    @pl.loop(0, WROWS // 8)
    def _(t):
        for j in range(8):
            pltpu.async_copy(ones_v, acc_sh.at[idx_v.at[t * 8 + j]], sem,
                             add=True)

    pltpu.make_async_copy(edges.at[1, pl.ds(0, WROWS)], idx_v, sem).wait()
    plsc.subcore_barrier()
    pltpu.sync_copy(acc_sh.at[pl.ds(sid * SLC, SLC)],
                    deg_out.at[cid, pl.ds(sid * SLC, SLC)])


@functools.lru_cache(maxsize=None)
def _hist_call():
    mesh = plsc.VectorSubcoreMesh(core_axis_name="c", subcore_axis_name="s",
                                  num_cores=NC, num_subcores=NS)
    return pl.kernel(
        _hist_body,
        out_type=jax.ShapeDtypeStruct((NC, RPAD), jnp.float32),
        mesh=mesh,
        scratch_types=[
            pltpu.VMEM((WROWS, 128), jnp.int32),
            pltpu.VMEM((128,), jnp.float32),
            pltpu.VMEM((SLC,), jnp.float32),
            pltpu.VMEM_SHARED((RPAD,), jnp.float32),
            pltpu.SemaphoreType.DMA,
        ],
    )


def _agg_body(edges, table, part_out,
              sidx_v, didx_v, rows_v, zer_v, table_sh, acc_sh,
              gsem0, gsem1, gsem2, gsem3, ssem):
    cid = lax.axis_index("c")
    sid = lax.axis_index("s")
    wid = sid * NC + cid
    wedge = WIN * 128
    gsems = [gsem0, gsem1, gsem2, gsem3]

    @pl.loop(0, SLC)
    def _(i):
        zer_v[i, :] = jnp.zeros((DHID,), jnp.float32)

    pltpu.sync_copy(zer_v, acc_sh.at[pl.ds(sid * SLC, SLC)])
    pltpu.sync_copy(table.at[pl.ds(sid * SLC, SLC)],
                    table_sh.at[pl.ds(sid * SLC, SLC)])
    pltpu.sync_copy(edges.at[0, pl.ds(wid * WROWS, WROWS)], sidx_v)
    pltpu.sync_copy(edges.at[1, pl.ds(wid * WROWS, WROWS)], didx_v)
    plsc.subcore_barrier()

    def fire_gathers(w, b):
        for j in range(WIN):
            pltpu.async_copy(table_sh.at[sidx_v.at[w * WIN + j]],
                             rows_v.at[b, pl.ds(j * 128, 128)], gsems[b])

    def drain(sem):
        pltpu.make_async_copy(part_out.at[0, pl.ds(0, wedge)],
                              rows_v.at[0], sem).wait()

    fire_gathers(0, 0)
    fire_gathers(1, 1)
    fire_gathers(2, 2)

    @pl.loop(0, NWIN // NBUF)
    def _(t):
        for k in range(NBUF):
            w = NBUF * t + k
            drain(gsems[k])
            for j in range(WIN):
                pltpu.async_copy(rows_v.at[k, pl.ds(j * 128, 128)],
                                 acc_sh.at[didx_v.at[w * WIN + j]], ssem,
                                 add=True)

            @pl.when(w >= 1)
            def _():
                drain(ssem)

            @pl.when(w + 3 <= NWIN - 1)
            def _():
                fire_gathers(w + 3, (k + 3) % NBUF)

    drain(ssem)
    plsc.subcore_barrier()
    pltpu.sync_copy(acc_sh.at[pl.ds(sid * SLC, SLC)],
                    part_out.at[cid, pl.ds(sid * SLC, SLC)])


@functools.lru_cache(maxsize=None)
def _agg_call():
    mesh = plsc.VectorSubcoreMesh(core_axis_name="c", subcore_axis_name="s",
                                  num_cores=NC, num_subcores=NS)
    return pl.kernel(
        _agg_body,
        out_type=jax.ShapeDtypeStruct((NC, RPAD, DHID), jnp.float32),
        mesh=mesh,
        compiler_params=pltpu.CompilerParams(use_tc_tiling_on_sc=False),
        scratch_types=[
            pltpu.VMEM((WROWS, 128), jnp.int32),
            pltpu.VMEM((WROWS, 128), jnp.int32),
            pltpu.VMEM((NBUF, WIN * 128, DHID), jnp.float32),
            pltpu.VMEM((SLC, DHID), jnp.float32),
            pltpu.VMEM_SHARED((RPAD, DHID), jnp.float32),
            pltpu.VMEM_SHARED((RPAD, DHID), jnp.float32),
            pltpu.SemaphoreType.DMA,
            pltpu.SemaphoreType.DMA,
            pltpu.SemaphoreType.DMA,
            pltpu.SemaphoreType.DMA,
            pltpu.SemaphoreType.DMA,
        ],
    )


def _tc0_body(x_ref, w1_ref, m_ref):
    m_ref[...] = jnp.dot(x_ref[...], w1_ref[...],
                         preferred_element_type=jnp.float32)


def _tc1_body(deg_ref, m_ref, hp_ref, dis_ref):
    d = deg_ref[...]
    dis = lax.rsqrt(d[0:1, :] + d[1:2, :] + 1.0)
    dis = dis.reshape(RPAD, 1)
    hp_ref[...] = m_ref[...] * dis
    dis_ref[...] = dis


def _tc2_body(p1a_ref, p1b_ref, hp_ref, dis_ref, b1_ref, g_ref):
    hp = hp_ref[...]
    agg = p1a_ref[0] + p1b_ref[0] + hp
    dis = dis_ref[...]
    pre = agg * dis + b1_ref[...]
    g_ref[...] = jnp.maximum(pre, 0.0) * dis


def _tc3_body(p2a_ref, p2b_ref, g_ref, dis_ref, w2_ref, b2_ref, out_ref):
    g = g_ref[...]
    dis = dis_ref[...]
    agg = (p2a_ref[0] + p2b_ref[0] + g) * dis
    z = jnp.dot(agg, w2_ref[...],
                preferred_element_type=jnp.float32) + b2_ref[...]
    m = jnp.max(z, axis=1, keepdims=True)
    lse = m + jnp.log(jnp.sum(jnp.exp(z - m), axis=1, keepdims=True))
    out_ref[...] = z - lse


def _row_spec(shape):
    return pl.BlockSpec((RPAD,) + shape, lambda i: (0,) + (0,) * len(shape))


def _part_spec(core):
    return pl.BlockSpec((1, RPAD, DHID), lambda i, c=core: (c, 0, 0))


def _full_spec(shape):
    return pl.BlockSpec(shape, lambda i: (0,) * len(shape))


_deg_spec = _full_spec((2, RPAD))

_tc0_call = pl.pallas_call(
    _tc0_body,
    grid=(1,),
    in_specs=[_row_spec((DIN,)), _full_spec((DIN, DHID))],
    out_specs=[_row_spec((DHID,))],
    out_shape=[jax.ShapeDtypeStruct((RPAD, DHID), jnp.float32)],
)

_tc1_call = pl.pallas_call(
    _tc1_body,
    grid=(1,),
    in_specs=[_deg_spec, _row_spec((DHID,))],
    out_specs=[_row_spec((DHID,)), _row_spec((1,))],
    out_shape=[
        jax.ShapeDtypeStruct((RPAD, DHID), jnp.float32),
        jax.ShapeDtypeStruct((RPAD, 1), jnp.float32),
    ],
)

_tc2_call = pl.pallas_call(
    _tc2_body,
    grid=(1,),
    in_specs=[_part_spec(0), _part_spec(1), _row_spec((DHID,)),
              _row_spec((1,)), _full_spec((1, DHID))],
    out_specs=[_row_spec((DHID,))],
    out_shape=[jax.ShapeDtypeStruct((RPAD, DHID), jnp.float32)],
)

_tc3_call = pl.pallas_call(
    _tc3_body,
    grid=(1,),
    in_specs=[_part_spec(0), _part_spec(1), _row_spec((DHID,)),
              _row_spec((1,)), _full_spec((DHID, NCLS)), _full_spec((1, NCLS))],
    out_specs=[_row_spec((NCLS,))],
    out_shape=[jax.ShapeDtypeStruct((RPAD, NCLS), jnp.float32)],
)


def kernel(x, edge_index, W1, b1, W2, b2):
    edges = jnp.concatenate(
        [edge_index.astype(jnp.int32).reshape(2, NREAL, 128),
         jnp.asarray(_EPAD_CONST)], axis=1)

    (m,) = _tc0_call(x, W1)
    deg = _hist_call()(edges)
    hp, dis = _tc1_call(deg, m)
    p1 = _agg_call()(edges, hp)
    (g,) = _tc2_call(p1, p1, hp, dis, b1.reshape(1, DHID))
    p2 = _agg_call()(edges, g)
    (out,) = _tc3_call(p2, p2, g, dis, W2, b2.reshape(1, NCLS))
    return out[:NNODES]


# speedup vs baseline: 1.0076x; 1.0076x over previous
import functools

import jax
import jax.numpy as jnp
from jax import lax
from jax.experimental import pallas as pl
from jax.experimental.pallas import tpu as pltpu
from jax.experimental.pallas import tpu_sc as plsc

NNODES = 10000
NEDGES = 320000
DIN = 128
DHID = 16
NCLS = 64

NC, NS = 2, 16
NW = NC * NS
RPAD = 10240
EROWS = 2560
NREAL = NEDGES // 128
WROWS = EROWS // NW
WIN = 4
NWIN = WROWS // WIN
NBUF = 4
SLC = RPAD // NS

import numpy as _np
_pi = _np.arange((EROWS - NREAL) * 128, dtype=_np.int32)
_EPAD_CONST = _np.stack([
    (_pi * 37) % NNODES,
    NNODES + _pi % (RPAD - NNODES),
]).reshape(2, EROWS - NREAL, 128)

_BLK = 512
_GRID = RPAD // _BLK


def _hist_body(edges, deg_out, idx_v, ones_v, zer_v, acc_sh, sem):
    cid = lax.axis_index("c")
    sid = lax.axis_index("s")
    wid = sid * NC + cid

    @pl.loop(0, 8)
    def _(i):
        ones_v[pl.ds(i * 16, 16)] = jnp.ones((16,), jnp.float32)

    @pl.loop(0, SLC // 16)
    def _(i):
        zer_v[pl.ds(i * 16, 16)] = jnp.zeros((16,), jnp.float32)

    pltpu.sync_copy(edges.at[1, pl.ds(wid * WROWS, WROWS)], idx_v)
    pltpu.sync_copy(zer_v, acc_sh.at[pl.ds(sid * SLC, SLC)])
    plsc.subcore_barrier()

    @pl.loop(0, WROWS // 8)
    def _(t):
        for j in range(8):
            pltpu.async_copy(ones_v, acc_sh.at[idx_v.at[t * 8 + j]], sem,
                             add=True)

    pltpu.make_async_copy(edges.at[1, pl.ds(0, WROWS)], idx_v, sem).wait()
    plsc.subcore_barrier()
    pltpu.sync_copy(acc_sh.at[pl.ds(sid * SLC, SLC)],
                    deg_out.at[cid, pl.ds(sid * SLC, SLC)])


@functools.lru_cache(maxsize=None)
def _hist_call():
    mesh = plsc.VectorSubcoreMesh(core_axis_name="c", subcore_axis_name="s",
                                  num_cores=NC, num_subcores=NS)
    return pl.kernel(
        _hist_body,
        out_type=jax.ShapeDtypeStruct((NC, RPAD), jnp.float32),
        mesh=mesh,
        scratch_types=[
            pltpu.VMEM((WROWS, 128), jnp.int32),
            pltpu.VMEM((128,), jnp.float32),
            pltpu.VMEM((SLC,), jnp.float32),
            pltpu.VMEM_SHARED((RPAD,), jnp.float32),
            pltpu.SemaphoreType.DMA,
        ],
    )


def _agg_body(edges, table, part_out,
              sidx_v, didx_v, rows_v, zer_v, table_sh, acc_sh,
              gsem0, gsem1, gsem2, gsem3, ssem):
    cid = lax.axis_index("c")
    sid = lax.axis_index("s")
    wid = sid * NC + cid
    wedge = WIN * 128
    gsems = [gsem0, gsem1, gsem2, gsem3]

    @pl.loop(0, SLC)
    def _(i):
        zer_v[i, :] = jnp.zeros((DHID,), jnp.float32)

    pltpu.sync_copy(zer_v, acc_sh.at[pl.ds(sid * SLC, SLC)])
    pltpu.sync_copy(table.at[pl.ds(sid * SLC, SLC)],
                    table_sh.at[pl.ds(sid * SLC, SLC)])
    pltpu.sync_copy(edges.at[0, pl.ds(wid * WROWS, WROWS)], sidx_v)
    pltpu.sync_copy(edges.at[1, pl.ds(wid * WROWS, WROWS)], didx_v)
    plsc.subcore_barrier()

    def fire_gathers(w, b):
        for j in range(WIN):
            pltpu.async_copy(table_sh.at[sidx_v.at[w * WIN + j]],
                             rows_v.at[b, pl.ds(j * 128, 128)], gsems[b])

    def drain(sem):
        pltpu.make_async_copy(part_out.at[0, pl.ds(0, wedge)],
                              rows_v.at[0], sem).wait()

    fire_gathers(0, 0)
    fire_gathers(1, 1)
    fire_gathers(2, 2)

    @pl.loop(0, NWIN // NBUF)
    def _(t):
        for k in range(NBUF):
            w = NBUF * t + k
            drain(gsems[k])
            for j in range(WIN):
                pltpu.async_copy(rows_v.at[k, pl.ds(j * 128, 128)],
                                 acc_sh.at[didx_v.at[w * WIN + j]], ssem,
                                 add=True)

            @pl.when(w >= 1)
            def _():
                drain(ssem)

            @pl.when(w + 3 <= NWIN - 1)
            def _():
                fire_gathers(w + 3, (k + 3) % NBUF)

    drain(ssem)
    plsc.subcore_barrier()
    pltpu.sync_copy(acc_sh.at[pl.ds(sid * SLC, SLC)],
                    part_out.at[cid, pl.ds(sid * SLC, SLC)])


@functools.lru_cache(maxsize=None)
def _agg_call():
    mesh = plsc.VectorSubcoreMesh(core_axis_name="c", subcore_axis_name="s",
                                  num_cores=NC, num_subcores=NS)
    return pl.kernel(
        _agg_body,
        out_type=jax.ShapeDtypeStruct((NC, RPAD, DHID), jnp.float32),
        mesh=mesh,
        compiler_params=pltpu.CompilerParams(use_tc_tiling_on_sc=False),
        scratch_types=[
            pltpu.VMEM((WROWS, 128), jnp.int32),
            pltpu.VMEM((WROWS, 128), jnp.int32),
            pltpu.VMEM((NBUF, WIN * 128, DHID), jnp.float32),
            pltpu.VMEM((SLC, DHID), jnp.float32),
            pltpu.VMEM_SHARED((RPAD, DHID), jnp.float32),
            pltpu.VMEM_SHARED((RPAD, DHID), jnp.float32),
            pltpu.SemaphoreType.DMA,
            pltpu.SemaphoreType.DMA,
            pltpu.SemaphoreType.DMA,
            pltpu.SemaphoreType.DMA,
            pltpu.SemaphoreType.DMA,
        ],
    )


def _tc1_body(deg_ref, x_ref, w1_ref, hp_ref, dis_ref):
    d = deg_ref[...]
    dis = lax.rsqrt(d[0:1, :] + d[1:2, :] + 1.0)
    dis = dis.reshape(RPAD, 1)
    m = jnp.dot(x_ref[...], w1_ref[...],
                preferred_element_type=jnp.float32)
    hp_ref[...] = m * dis
    dis_ref[...] = dis


def _tc2_body(p1a_ref, p1b_ref, hp_ref, dis_ref, b1_ref, g_ref):
    hp = hp_ref[...]
    agg = p1a_ref[0] + p1b_ref[0] + hp
    dis = dis_ref[...]
    pre = agg * dis + b1_ref[...]
    g_ref[...] = jnp.maximum(pre, 0.0) * dis


def _tc3_body(p2a_ref, p2b_ref, g_ref, dis_ref, w2_ref, b2_ref, out_ref):
    g = g_ref[...]
    dis = dis_ref[...]
    agg = (p2a_ref[0] + p2b_ref[0] + g) * dis
    z = jnp.dot(agg, w2_ref[...],
                preferred_element_type=jnp.float32) + b2_ref[...]
    m = jnp.max(z, axis=1, keepdims=True)
    lse = m + jnp.log(jnp.sum(jnp.exp(z - m), axis=1, keepdims=True))
    out_ref[...] = z - lse


def _row_spec(shape):
    return pl.BlockSpec((RPAD,) + shape, lambda i: (0,) + (0,) * len(shape))


def _part_spec(core):
    return pl.BlockSpec((1, RPAD, DHID), lambda i, c=core: (c, 0, 0))


def _full_spec(shape):
    return pl.BlockSpec(shape, lambda i: (0,) * len(shape))


_deg_spec = _full_spec((2, RPAD))

_tc1_call = pl.pallas_call(
    _tc1_body,
    grid=(1,),
    in_specs=[_deg_spec, _row_spec((DIN,)), _full_spec((DIN, DHID))],
    out_specs=[_row_spec((DHID,)), _row_spec((1,))],
    out_shape=[
        jax.ShapeDtypeStruct((RPAD, DHID), jnp.float32),
        jax.ShapeDtypeStruct((RPAD, 1), jnp.float32),
    ],
)

_tc2_call = pl.pallas_call(
    _tc2_body,
    grid=(1,),
    in_specs=[_part_spec(0), _part_spec(1), _row_spec((DHID,)),
              _row_spec((1,)), _full_spec((1, DHID))],
    out_specs=[_row_spec((DHID,))],
    out_shape=[jax.ShapeDtypeStruct((RPAD, DHID), jnp.float32)],
)

_tc3_call = pl.pallas_call(
    _tc3_body,
    grid=(1,),
    in_specs=[_part_spec(0), _part_spec(1), _row_spec((DHID,)),
              _row_spec((1,)), _full_spec((DHID, NCLS)), _full_spec((1, NCLS))],
    out_specs=[_row_spec((NCLS,))],
    out_shape=[jax.ShapeDtypeStruct((RPAD, NCLS), jnp.float32)],
)


def kernel(x, edge_index, W1, b1, W2, b2):
    edges = jnp.concatenate(
        [edge_index.astype(jnp.int32).reshape(2, NREAL, 128),
         jnp.asarray(_EPAD_CONST)], axis=1)

    deg = _hist_call()(edges)
    hp, dis = _tc1_call(deg, x, W1)
    p1 = _agg_call()(edges, hp)
    (g,) = _tc2_call(p1, p1, hp, dis, b1.reshape(1, DHID))
    p2 = _agg_call()(edges, g)
    (out,) = _tc3_call(p2, p2, g, dis, W2, b2.reshape(1, NCLS))
    return out[:NNODES]
